# Initial kernel scaffold; baseline (speedup 1.0000x reference)
import jax
import jax.numpy as jnp
from jax.experimental import pallas as pl


def kernel(cluster_assignments, n_clusters):
    raise NotImplementedError("write your pallas kernel here")


# speedup vs baseline: 2.2401x; 2.2401x over previous
import functools

import jax
import jax.numpy as jnp
from jax import lax
from jax.experimental import pallas as pl
from jax.experimental.pallas import tpu as pltpu
from jax.experimental.pallas import tpu_sc as plsc

N_TOTAL = 16777216
N_BINS = 1024
NC = 2
NS = 16
L = 16
NW = NC * NS
PER_W = N_TOTAL // NW
BLK = 32768
NBLK = PER_W // BLK
NPAIR = NBLK // 2

_mesh = plsc.VectorSubcoreMesh(core_axis_name="c", subcore_axis_name="s")


@functools.partial(
    pl.kernel,
    out_type=jax.ShapeDtypeStruct((NW, N_BINS), jnp.float32),
    mesh=_mesh,
    scratch_types=[
        pltpu.VMEM((BLK,), jnp.int32),
        pltpu.VMEM((BLK,), jnp.int32),
        pltpu.VMEM((L * N_BINS,), jnp.float32),
        pltpu.VMEM((N_BINS,), jnp.float32),
        pltpu.SemaphoreType.DMA,
        pltpu.SemaphoreType.DMA,
    ],
    compiler_params=pltpu.CompilerParams(needs_layout_passes=False),
)
def _sc_histogram(x_hbm, out_hbm, buf0, buf1, hists, hist1, sem0, sem1):
    wid = lax.axis_index("s") * NC + lax.axis_index("c")
    base = wid * PER_W

    pltpu.async_copy(x_hbm.at[pl.ds(base, BLK)], buf0, sem0)
    pltpu.async_copy(x_hbm.at[pl.ds(base + BLK, BLK)], buf1, sem1)

    zeros16 = jnp.zeros((L,), jnp.float32)

    def _zero(i, c):
        hists[pl.ds(i * L, L)] = zeros16
        return c

    lax.fori_loop(0, (L * N_BINS) // L, _zero, 0, unroll=8)

    lane_off = lax.iota(jnp.int32, L) * N_BINS
    ones16 = jnp.ones((L,), jnp.float32)

    def _scatter_block(buf):
        def _s(i, c):
            idx = buf[pl.ds(i * L, L)] + lane_off
            plsc.addupdate_scatter(hists, [idx], ones16)
            return c

        lax.fori_loop(0, BLK // L, _s, 0, unroll=8)

    def _pair(g, c):
        pltpu.make_async_copy(x_hbm.at[pl.ds(0, BLK)], buf0, sem0).wait()
        _scatter_block(buf0)

        @pl.when(g < NPAIR - 1)
        def _():
            pltpu.async_copy(
                x_hbm.at[pl.ds(base + (2 * g + 2) * BLK, BLK)], buf0, sem0)

        pltpu.make_async_copy(x_hbm.at[pl.ds(0, BLK)], buf1, sem1).wait()
        _scatter_block(buf1)

        @pl.when(g < NPAIR - 1)
        def _():
            pltpu.async_copy(
                x_hbm.at[pl.ds(base + (2 * g + 3) * BLK, BLK)], buf1, sem1)

        return c

    lax.fori_loop(0, NPAIR, _pair, 0)

    def _red(g, c):
        acc = hists[pl.ds(g * L, L)]
        for l in range(1, L):
            acc = acc + hists[pl.ds(l * N_BINS + g * L, L)]
        hist1[pl.ds(g * L, L)] = acc
        return c

    lax.fori_loop(0, N_BINS // L, _red, 0)

    pltpu.sync_copy(hist1, out_hbm.at[wid])


def _entropy_body(counts_ref, out_ref):
    c = counts_ref[...]
    counts = jnp.sum(c, axis=0, keepdims=True)
    total = jnp.sum(counts)
    p = counts / total
    out_ref[0, 0] = -jnp.sum(p * jnp.log(p + 1e-08))


_entropy_tc = pl.pallas_call(
    _entropy_body,
    out_shape=jax.ShapeDtypeStruct((1, 1), jnp.float32),
    in_specs=[pl.BlockSpec(memory_space=pltpu.VMEM)],
    out_specs=pl.BlockSpec(memory_space=pltpu.SMEM),
)


def kernel(cluster_assignments, n_clusters):
    counts = _sc_histogram(cluster_assignments)
    return _entropy_tc(counts)[0, 0]


# speedup vs baseline: 6.5311x; 2.9156x over previous
import functools

import jax
import jax.numpy as jnp
from jax import lax
from jax.experimental import pallas as pl
from jax.experimental.pallas import tpu as pltpu
from jax.experimental.pallas import tpu_sc as plsc

N_TOTAL = 16777216
N_BINS = 1024
NC = 2
NS = 16
L = 16
NW = NC * NS
PER_W = N_TOTAL // NW
BLK = 32768
NBLK = PER_W // BLK
NPAIR = NBLK // 2

_mesh = plsc.VectorSubcoreMesh(core_axis_name="c", subcore_axis_name="s")


@functools.partial(
    pl.kernel,
    out_type=jax.ShapeDtypeStruct((NW, N_BINS), jnp.float32),
    mesh=_mesh,
    scratch_types=[
        pltpu.VMEM((BLK,), jnp.int32),
        pltpu.VMEM((BLK,), jnp.int32),
        pltpu.VMEM((L * N_BINS,), jnp.float32),
        pltpu.VMEM((N_BINS,), jnp.float32),
        pltpu.SemaphoreType.DMA,
        pltpu.SemaphoreType.DMA,
    ],
    compiler_params=pltpu.CompilerParams(needs_layout_passes=False),
)
def _sc_histogram(x_hbm, out_hbm, buf0, buf1, hists, hist1, sem0, sem1):
    wid = lax.axis_index("s") * NC + lax.axis_index("c")
    base = wid * PER_W

    pltpu.async_copy(x_hbm.at[pl.ds(base, BLK)], buf0, sem0)
    pltpu.async_copy(x_hbm.at[pl.ds(base + BLK, BLK)], buf1, sem1)

    zeros16 = jnp.zeros((L,), jnp.float32)

    @plsc.parallel_loop(0, (L * N_BINS) // L, unroll=8)
    def _zero(i):
        hists[pl.ds(i * L, L)] = zeros16

    lane_off = lax.iota(jnp.int32, L) * N_BINS
    ones16 = jnp.ones((L,), jnp.float32)

    def _scatter_block(buf):
        @plsc.parallel_loop(0, BLK // L, unroll=8)
        def _s(i):
            idx = buf[pl.ds(i * L, L)] + lane_off
            plsc.addupdate_scatter(hists, [idx], ones16)

    def _pair(g, c):
        pltpu.make_async_copy(x_hbm.at[pl.ds(0, BLK)], buf0, sem0).wait()
        _scatter_block(buf0)

        @pl.when(g < NPAIR - 1)
        def _():
            pltpu.async_copy(
                x_hbm.at[pl.ds(base + (2 * g + 2) * BLK, BLK)], buf0, sem0)

        pltpu.make_async_copy(x_hbm.at[pl.ds(0, BLK)], buf1, sem1).wait()
        _scatter_block(buf1)

        @pl.when(g < NPAIR - 1)
        def _():
            pltpu.async_copy(
                x_hbm.at[pl.ds(base + (2 * g + 3) * BLK, BLK)], buf1, sem1)

        return c

    lax.fori_loop(0, NPAIR, _pair, 0)

    @plsc.parallel_loop(0, N_BINS // L, unroll=2)
    def _red(g):
        acc = hists[pl.ds(g * L, L)]
        for l in range(1, L):
            acc = acc + hists[pl.ds(l * N_BINS + g * L, L)]
        hist1[pl.ds(g * L, L)] = acc

    pltpu.sync_copy(hist1, out_hbm.at[wid])


def _entropy_body(counts_ref, out_ref):
    c = counts_ref[...]
    counts = jnp.sum(c, axis=0, keepdims=True)
    total = jnp.sum(counts)
    p = counts / total
    out_ref[0, 0] = -jnp.sum(p * jnp.log(p + 1e-08))


_entropy_tc = pl.pallas_call(
    _entropy_body,
    out_shape=jax.ShapeDtypeStruct((1, 1), jnp.float32),
    in_specs=[pl.BlockSpec(memory_space=pltpu.VMEM)],
    out_specs=pl.BlockSpec(memory_space=pltpu.SMEM),
)


def kernel(cluster_assignments, n_clusters):
    counts = _sc_histogram(cluster_assignments)
    return _entropy_tc(counts)[0, 0]
